# Initial kernel scaffold; baseline (speedup 1.0000x reference)
import jax
import jax.numpy as jnp
from jax.experimental import pallas as pl


def kernel(pred, target, class_weights):
    raise NotImplementedError("write your pallas kernel here")


# speedup vs baseline: 34.3205x; 34.3205x over previous
import functools

import jax
import jax.numpy as jnp
from jax import lax
from jax.experimental import pallas as pl
from jax.experimental.pallas import tpu as pltpu
from jax.experimental.pallas import tpu_sc as plsc

NBINS = 4096
EMAX = 8.0
SCALE = NBINS / EMAX
NLANE = 16
TBL = NLANE * NBINS
NWORK = 32
NCLS = 3
PIX = 512 * 512
NBATCH = 16
CHUNK = NBATCH * PIX // NWORK
BLK = 16384
NBLK = CHUNK // BLK
PACK_SHIFT = 14


def _sc_hist_body(pred_hbm, target_hbm, out_hbm, table, pbuf, tbuf):
    wid = lax.axis_index("s") * 2 + lax.axis_index("c")
    run = wid // 2
    half = wid % 2
    lane = lax.iota(jnp.int32, NLANE)
    lanebase = lane * NBINS + (NBINS - 1)
    zero16 = jnp.zeros((NLANE,), jnp.int32)

    for c in range(NCLS):
        def zero_body(j, _):
            table[pl.ds(j * NLANE, NLANE)] = zero16
            return 0

        lax.fori_loop(0, TBL // NLANE, zero_body, 0)

        base = (run * NCLS + c) * PIX + half * CHUNK

        def blk_body(b, _):
            off = base + b * BLK
            pltpu.sync_copy(pred_hbm.at[pl.ds(off, BLK)], pbuf)
            pltpu.sync_copy(target_hbm.at[pl.ds(off, BLK)], tbuf)

            def vec_body(i, _):
                p = pbuf[pl.ds(i * NLANE, NLANE)]
                t = tbuf[pl.ds(i * NLANE, NLANE)]
                e = 1.0 - p * (2.0 * t - 1.0)
                bi = jnp.clip((e * SCALE).astype(jnp.int32), 0, NBINS - 1)
                idx = lanebase - bi
                pv = t.astype(jnp.int32) * (1 << PACK_SHIFT) + 1
                plsc.addupdate_scatter(table, [idx], pv)
                return 0

            lax.fori_loop(0, BLK // NLANE, vec_body, 0)
            return 0

        lax.fori_loop(0, NBLK, blk_body, 0)
        pltpu.sync_copy(table, out_hbm.at[pl.ds((c * NWORK + wid) * TBL, TBL)])


def _sc_hist(pred_flat, target_flat):
    mesh = plsc.VectorSubcoreMesh(core_axis_name="c", subcore_axis_name="s")
    return pl.kernel(
        _sc_hist_body,
        out_type=jax.ShapeDtypeStruct((NCLS * NWORK * TBL,), jnp.int32),
        mesh=mesh,
        compiler_params=pltpu.CompilerParams(needs_layout_passes=False),
        scratch_types=[
            pltpu.VMEM((TBL,), jnp.int32),
            pltpu.VMEM((BLK,), jnp.float32),
            pltpu.VMEM((BLK,), jnp.float32),
        ],
    )(pred_flat, target_flat)


def _shift_lanes(x, s):
    pad = jnp.zeros((x.shape[0], s), x.dtype)
    return jnp.concatenate([pad, x[:, : x.shape[1] - s]], axis=1)


def _shift_rows(x, s):
    pad = jnp.zeros((s, x.shape[1]), x.dtype)
    return jnp.concatenate([pad, x[: x.shape[0] - s]], axis=0)


def _finalize_body(tbl_ref, w_ref, out_ref):
    total = jnp.float32(0.0)
    rows = NWORK * NLANE
    step = 8
    for c in range(NCLS):
        def red_body(k, acc):
            cnt_a, pos_a = acc
            chunk = tbl_ref[c, pl.ds(k * step, step)]
            cnt_a = cnt_a + jnp.sum(chunk & ((1 << PACK_SHIFT) - 1), axis=0)
            pos_a = pos_a + jnp.sum(chunk >> PACK_SHIFT, axis=0)
            return (cnt_a, pos_a)

        zero = jnp.zeros((32, 128), jnp.int32)
        cnt, pos = lax.fori_loop(0, rows // step, red_body, (zero, zero))

        ncum, pcum = cnt, pos
        for s in (1, 2, 4, 8, 16, 32, 64):
            ncum = ncum + _shift_lanes(ncum, s)
            pcum = pcum + _shift_lanes(pcum, s)
        nrt = ncum[:, 127:128]
        prt = pcum[:, 127:128]
        ncar, pcar = nrt, prt
        for s in (1, 2, 4, 8, 16):
            ncar = ncar + _shift_rows(ncar, s)
            pcar = pcar + _shift_rows(pcar, s)
        ncum = ncum + (ncar - nrt)
        pcum = pcum + (pcar - prt)

        g = jnp.sum(pos)
        nf = ncum.astype(jnp.float32)
        pf = pcum.astype(jnp.float32)
        gf = g.astype(jnp.float32)
        denom = jnp.maximum(gf + nf - pf, 1.0)
        jac = jnp.where(ncum == 0, 0.0, 1.0 - (gf - pf) / denom)

        bi = (
            lax.broadcasted_iota(jnp.int32, (32, 128), 0) * 128
            + lax.broadcasted_iota(jnp.int32, (32, 128), 1)
        )
        coef = jnp.where(
            bi == NBINS - 1, 0.0, jnp.where(bi == NBINS - 2, 1.5, 1.0)
        ) / SCALE
        loss_c = jnp.sum(jac * coef)
        total = total + w_ref[c] * loss_c
    out_ref[0, 0] = total


def _tc_finalize(tbl, weights):
    return pl.pallas_call(
        _finalize_body,
        out_shape=jax.ShapeDtypeStruct((1, 1), jnp.float32),
        in_specs=[
            pl.BlockSpec(memory_space=pltpu.VMEM),
            pl.BlockSpec(memory_space=pltpu.SMEM),
        ],
        out_specs=pl.BlockSpec(memory_space=pltpu.SMEM),
    )(tbl, weights)


@jax.jit
def kernel(pred, target, class_weights):
    pred_flat = pred.reshape(-1)
    target_flat = target.reshape(-1)
    tables = _sc_hist(pred_flat, target_flat)
    tbl = tables.reshape(NCLS, NWORK * NLANE, 32, 128)
    out = _tc_finalize(tbl, class_weights)
    return out.reshape(())
